# Initial kernel scaffold; baseline (speedup 1.0000x reference)
import jax
import jax.numpy as jnp
from jax.experimental import pallas as pl


def kernel(x, edge_attr, e_W1, e_b1, e_W2, e_b2, n_W1, n_b1, n_W2, n_b2, o_W1, o_b1, o_W2, o_b2, o_W3, o_b3, o_W4, o_b4, edge_index, batch):
    raise NotImplementedError("write your pallas kernel here")


# speedup vs baseline: 6.8109x; 6.8109x over previous
import functools

import jax
import jax.numpy as jnp
import numpy as np
from jax import lax
from jax.experimental import pallas as pl
from jax.experimental.pallas import tpu as pltpu
from jax.experimental.pallas import tpu_sc as plsc

_NC = 2
_NS = 16
_NW = _NC * _NS

_LOG_A = np.float32(np.log(0.7660379))
_LOG_B = np.float32(np.log(0.12117091))
_LOG_C = np.float32(np.log(1.2125463))
_LOG_D = np.float32(np.log(0.1562228))


@functools.lru_cache(maxsize=None)
def _edge_sc_kernel(N, E, N_pad, CH):
    EW = E // _NW
    NCH = EW // CH
    NP16 = N_pad // _NS

    mesh = plsc.VectorSubcoreMesh(core_axis_name="c", subcore_axis_name="s")

    pieces = [(i * CH, CH) for i in range(NP16 // CH)]
    if NP16 % CH:
        pieces.append((NP16 - NP16 % CH, NP16 % CH))

    def body(x_hbm, ei_hbm, attr_hbm, out_hbm,
             x_v, row_v, col_v, attr_v, e0_v, e1_v,
             agg0_s, agg1_s):
        c = lax.axis_index("c")
        s = lax.axis_index("s")
        wid = c * _NS + s

        def zbody(i, carry):
            e0_v[pl.ds(i * 16, 16)] = jnp.zeros((16,), jnp.float32)
            return carry
        lax.fori_loop(0, CH // 16, zbody, 0)
        for off, ln in pieces:
            pltpu.sync_copy(e0_v.at[pl.ds(0, ln)],
                            agg0_s.at[pl.ds(s * NP16 + off, ln)])
            pltpu.sync_copy(e0_v.at[pl.ds(0, ln)],
                            agg1_s.at[pl.ds(s * NP16 + off, ln)])

        pltpu.sync_copy(x_hbm, x_v)
        plsc.subcore_barrier()

        iota16 = lax.iota(jnp.int32, 16)

        def chunk_body(ci, carry):
            base = wid * EW + ci * CH
            pltpu.sync_copy(ei_hbm.at[pl.ds(base, CH)], row_v)
            pltpu.sync_copy(ei_hbm.at[pl.ds(E + base, CH)], col_v)
            pltpu.sync_copy(attr_hbm.at[pl.ds(base * 3, CH * 3)], attr_v)

            def vec_body(k, vcarry):
                p = k * 16
                rows = row_v[pl.ds(p, 16)]
                cols = col_v[pl.ds(p, 16)]
                vi = plsc.load_gather(x_v, [rows])
                vj = plsc.load_gather(x_v, [cols])
                pos3 = (p + iota16) * 3
                d = plsc.load_gather(attr_v, [pos3])
                b = plsc.load_gather(attr_v, [pos3 + 2])
                e0 = (jnp.abs((vi / 0.9484139 - (vj - 0.2123214)) * -1.3248432)
                      + (d - 1.7348461 + b + vj) * -0.12084719)
                e1 = (jnp.abs((vi - vj * 1.0584362) * 1.5344211 + 0.45368108)
                      + (vi - vj * 1.0239582) * 1.931712 + 0.546892)
                e0_v[pl.ds(p, 16)] = e0
                e1_v[pl.ds(p, 16)] = e1
                return vcarry
            lax.fori_loop(0, CH // 16, vec_body, 0)

            pltpu.sync_copy(e0_v, agg0_s.at[col_v], add=True)
            pltpu.sync_copy(e1_v, agg1_s.at[col_v], add=True)
            return carry
        lax.fori_loop(0, NCH, chunk_body, 0)

        plsc.subcore_barrier()
        obase = c * 2 * N_pad + s * NP16
        for off, ln in pieces:
            pltpu.sync_copy(agg0_s.at[pl.ds(s * NP16 + off, ln)],
                            e0_v.at[pl.ds(0, ln)])
            pltpu.sync_copy(e0_v.at[pl.ds(0, ln)],
                            out_hbm.at[pl.ds(obase + off, ln)])
            pltpu.sync_copy(agg1_s.at[pl.ds(s * NP16 + off, ln)],
                            e1_v.at[pl.ds(0, ln)])
            pltpu.sync_copy(e1_v.at[pl.ds(0, ln)],
                            out_hbm.at[pl.ds(obase + N_pad + off, ln)])

    return pl.kernel(
        body,
        out_type=jax.ShapeDtypeStruct((_NC * 2 * N_pad,), jnp.float32),
        mesh=mesh,
        scratch_types=[
            pltpu.VMEM((N,), jnp.float32),
            pltpu.VMEM((CH,), jnp.int32),
            pltpu.VMEM((CH,), jnp.int32),
            pltpu.VMEM((CH * 3,), jnp.float32),
            pltpu.VMEM((CH,), jnp.float32),
            pltpu.VMEM((CH,), jnp.float32),
            pltpu.VMEM_SHARED((N_pad,), jnp.float32),
            pltpu.VMEM_SHARED((N_pad,), jnp.float32),
        ],
        compiler_params=pltpu.CompilerParams(needs_layout_passes=False),
        name="edge_scatter_sc",
    )


def _node_tc_body(N, R,
                  x_ref, agg_ref, w1, b1, w2, b2, w3, b3, w4, b4, out_ref):
    xv = x_ref[...]
    s1 = agg_ref[0] + agg_ref[2]
    s2 = agg_ref[1] + agg_ref[3]
    gidx = (lax.broadcasted_iota(jnp.int32, (R, 128), 0) * 128
            + lax.broadcasted_iota(jnp.int32, (R, 128), 1))
    mask = gidx < N

    n1 = ((jnp.exp((s2 / 0.3038425 + s1) * _LOG_A)
           + jnp.exp(s1 * _LOG_B) / -0.7256157)
          * jnp.exp(xv * _LOG_C) + 0.12262904)
    t = s2 + (s1 + -3.283101 - xv / 0.79082423) * 0.31992579
    n1_n2 = 0.7872602 - jnp.sqrt(jnp.log(jnp.exp(t * _LOG_D) + 1.4462701))
    h0 = jnp.where(mask, n1, 0.0)
    h1 = jnp.where(mask, n1_n2 - n1, 0.0)
    ps1 = jnp.sum(h0)
    ps2 = jnp.sum(h1)

    a = jnp.maximum(ps1 * w1[0, 0] + ps2 * w1[1, 0] + b1[0], 0.0)
    b_ = jnp.maximum(ps1 * w1[0, 1] + ps2 * w1[1, 1] + b1[1], 0.0)
    a2 = jnp.maximum(a * w2[0, 0] + b_ * w2[1, 0] + b2[0], 0.0)
    b2_ = jnp.maximum(a * w2[0, 1] + b_ * w2[1, 1] + b2[1], 0.0)
    a3 = jnp.maximum(a2 * w3[0, 0] + b2_ * w3[1, 0] + b3[0], 0.0)
    b3_ = jnp.maximum(a2 * w3[0, 1] + b2_ * w3[1, 1] + b3[1], 0.0)
    o1 = a3 * w4[0, 1] + b3_ * w4[1, 1] + b4[1]
    o0 = ((ps2 / -0.18032177 + ps1 * 2.2054937
           + jnp.abs(ps2 * 0.9565731 + ps1 * 0.8225316))
          * 0.00046277698 + -0.24634261)

    r8 = lax.broadcasted_iota(jnp.int32, (8, 128), 0)
    c8 = lax.broadcasted_iota(jnp.int32, (8, 128), 1)
    out_ref[...] = jnp.where(
        (r8 == 0) & (c8 == 0), o0,
        jnp.where((r8 == 0) & (c8 == 1), o1, 0.0))


@functools.lru_cache(maxsize=None)
def _node_tc_kernel(N, R):
    smem = pl.BlockSpec(memory_space=pltpu.SMEM)
    return pl.pallas_call(
        functools.partial(_node_tc_body, N, R),
        out_shape=jax.ShapeDtypeStruct((8, 128), jnp.float32),
        in_specs=[pl.BlockSpec(memory_space=pltpu.VMEM),
                  pl.BlockSpec(memory_space=pltpu.VMEM),
                  smem, smem, smem, smem, smem, smem, smem, smem],
        out_specs=pl.BlockSpec(memory_space=pltpu.VMEM),
        name="node_pool_tc",
    )


def kernel(x, edge_attr, e_W1, e_b1, e_W2, e_b2, n_W1, n_b1, n_W2, n_b2,
           o_W1, o_b1, o_W2, o_b2, o_W3, o_b3, o_W4, o_b4, edge_index, batch):
    N = x.shape[0]
    E = edge_index.shape[1]
    N_pad = ((N + 127) // 128) * 128
    EW = E // _NW
    CH = 2000 if EW % 2000 == 0 else 1000
    x_flat = x[:, 0]

    aggs = _edge_sc_kernel(N, E, N_pad, CH)(
        x_flat, edge_index.reshape(2 * E), edge_attr.reshape(3 * E))

    R = N_pad // 128
    x_pad = jnp.pad(x_flat, (0, N_pad - N)).reshape(R, 128)
    agg4 = aggs.reshape(4, R, 128)
    out8 = _node_tc_kernel(N, R)(
        x_pad, agg4, o_W1, o_b1, o_W2, o_b2, o_W3, o_b3, o_W4, o_b4)
    return out8[0:1, 0:2]


# speedup vs baseline: 384.6825x; 56.4808x over previous
import functools

import jax
import jax.numpy as jnp
import numpy as np
from jax import lax
from jax.experimental import pallas as pl
from jax.experimental.pallas import tpu as pltpu
from jax.experimental.pallas import tpu_sc as plsc

_NC = 2
_NS = 16
_NW = _NC * _NS
_BLK = 128
_NBLK = 8
_CH = _NBLK * _BLK

_LOG_A = np.float32(np.log(0.7660379))
_LOG_B = np.float32(np.log(0.12117091))
_LOG_C = np.float32(np.log(1.2125463))
_LOG_D = np.float32(np.log(0.1562228))


@functools.lru_cache(maxsize=None)
def _edge_sc_kernel(N, E, N_pad):
    NB = E // _BLK
    q, r = divmod(NB, _NW)
    NCH = -(-(q + (1 if r else 0)) // _NBLK)
    assert NCH % 2 == 0 and NCH >= 4, NCH
    NP16 = N_pad // _NS

    mesh = plsc.VectorSubcoreMesh(core_axis_name="c", subcore_axis_name="s")

    pieces = [(i * _CH, _CH) for i in range(NP16 // _CH)]
    if NP16 % _CH:
        pieces.append((NP16 - NP16 % _CH, NP16 % _CH))

    def body(x_hbm, ei_hbm, db_hbm, out_hbm,
             x_v, ei_a, db_a, col_a, e0_a, e1_a,
             ei_b, db_b, col_b, e0_b, e1_b,
             agg0_s, agg1_s, in_sem_a, in_sem_b, sc_sem_a, sc_sem_b):
        c = lax.axis_index("c")
        s = lax.axis_index("s")
        wid = c * _NS + s
        start_blk = q * wid + jnp.minimum(wid, r)
        nblk_w = q + jnp.where(wid < r, 1, 0)
        last_base = start_blk + nblk_w - _NBLK
        seta = (ei_a, db_a, col_a, e0_a, e1_a, in_sem_a, sc_sem_a)
        setb = (ei_b, db_b, col_b, e0_b, e1_b, in_sem_b, sc_sem_b)

        def chunk_base(ci):
            return jnp.minimum(start_blk + ci * _NBLK, last_base)

        def fire_inputs(ci, st):
            ei_v, db_v, _, _, _, isem, _ = st
            base = chunk_base(ci)
            pltpu.async_copy(ei_hbm.at[pl.ds(base, _NBLK), :, :], ei_v, isem)
            pltpu.async_copy(db_hbm.at[pl.ds(base * _BLK, _CH)], db_v, isem)

        def wait_inputs(ci, st):
            ei_v, db_v, _, _, _, isem, _ = st
            base = chunk_base(ci)
            pltpu.make_async_copy(
                ei_hbm.at[pl.ds(base, _NBLK), :, :], ei_v, isem).wait()
            pltpu.make_async_copy(
                db_hbm.at[pl.ds(base * _BLK, _CH)], db_v, isem).wait()

        def compute(ci, st):
            ei_v, db_v, col_v, e0_v, e1_v, _, _ = st
            base = chunk_base(ci)
            vstart = start_blk + ci * _NBLK

            @plsc.parallel_loop(0, _CH, step=16)
            def vec_body(p):
                blk = p // _BLK
                l = p - blk * _BLK
                rows = ei_v[blk, 0, pl.ds(l, 16)]
                cols = ei_v[blk, 1, pl.ds(l, 16)]
                vi = plsc.load_gather(x_v, [rows])
                vj = plsc.load_gather(x_v, [cols])
                db = db_v[pl.ds(p, 16)]
                valid = base + blk >= vstart
                e0 = (jnp.abs((vi / 0.9484139 - (vj - 0.2123214)) * -1.3248432)
                      + (db - 1.7348461 + vj) * -0.12084719)
                e1 = (jnp.abs((vi - vj * 1.0584362) * 1.5344211 + 0.45368108)
                      + (vi - vj * 1.0239582) * 1.931712 + 0.546892)
                col_v[pl.ds(p, 16)] = cols
                e0_v[pl.ds(p, 16)] = jnp.where(valid, e0, 0.0)
                e1_v[pl.ds(p, 16)] = jnp.where(valid, e1, 0.0)

        def fire_scatter(st):
            _, _, col_v, e0_v, e1_v, _, ssem = st
            pltpu.async_copy(e0_v, agg0_s.at[col_v], ssem, add=True)
            pltpu.async_copy(e1_v, agg1_s.at[col_v], ssem, add=True)

        def drain_scatter(st):
            _, _, col_v, e0_v, e1_v, _, ssem = st
            pltpu.make_async_copy(e0_v, agg0_s.at[col_v], ssem).wait()
            pltpu.make_async_copy(e1_v, agg1_s.at[col_v], ssem).wait()

        def zbody(i, carry):
            e0_a[pl.ds(i * 16, 16)] = jnp.zeros((16,), jnp.float32)
            return carry
        lax.fori_loop(0, _CH // 16, zbody, 0)
        pltpu.async_copy(x_hbm, x_v, in_sem_a)
        fire_inputs(0, seta)
        fire_inputs(1, setb)
        for off, ln in pieces:
            pltpu.async_copy(e0_a.at[pl.ds(0, ln)],
                             agg0_s.at[pl.ds(s * NP16 + off, ln)], sc_sem_a)
            pltpu.async_copy(e0_a.at[pl.ds(0, ln)],
                             agg1_s.at[pl.ds(s * NP16 + off, ln)], sc_sem_a)
        for off, ln in pieces:
            pltpu.make_async_copy(
                e0_a.at[pl.ds(0, ln)],
                agg0_s.at[pl.ds(s * NP16 + off, ln)], sc_sem_a).wait()
            pltpu.make_async_copy(
                e0_a.at[pl.ds(0, ln)],
                agg1_s.at[pl.ds(s * NP16 + off, ln)], sc_sem_a).wait()
        plsc.subcore_barrier()

        pltpu.make_async_copy(x_hbm, x_v, in_sem_a).wait()
        wait_inputs(0, seta)
        compute(0, seta)
        fire_scatter(seta)

        def pair_body(g, carry):
            ci1 = 1 + 2 * g
            wait_inputs(ci1, setb)
            compute(ci1, setb)
            drain_scatter(seta)
            fire_inputs(ci1 + 1, seta)
            fire_scatter(setb)
            ci2 = ci1 + 1
            wait_inputs(ci2, seta)
            compute(ci2, seta)
            drain_scatter(setb)
            fire_inputs(ci2 + 1, setb)
            fire_scatter(seta)
            return carry
        lax.fori_loop(0, (NCH - 2) // 2, pair_body, 0)

        ci = NCH - 1
        wait_inputs(ci, setb)
        compute(ci, setb)
        drain_scatter(seta)
        fire_scatter(setb)
        drain_scatter(setb)

        plsc.subcore_barrier()
        obase = c * 2 * N_pad + s * NP16
        stage = [e0_a, e1_a, db_a, e0_b, e1_b, db_b]
        tasks = [(p_, off, ln) for p_ in (0, 1) for off, ln in pieces]
        for i0 in range(0, len(tasks), len(stage)):
            batch = list(zip(tasks[i0:i0 + len(stage)], stage))
            for (p_, off, ln), buf in batch:
                src = (agg0_s if p_ == 0 else agg1_s)
                pltpu.async_copy(src.at[pl.ds(s * NP16 + off, ln)],
                                 buf.at[pl.ds(0, ln)], in_sem_a)
            for (p_, off, ln), buf in batch:
                src = (agg0_s if p_ == 0 else agg1_s)
                pltpu.make_async_copy(src.at[pl.ds(s * NP16 + off, ln)],
                                      buf.at[pl.ds(0, ln)], in_sem_a).wait()
            for (p_, off, ln), buf in batch:
                dst = out_hbm.at[pl.ds(obase + p_ * N_pad + off, ln)]
                pltpu.async_copy(buf.at[pl.ds(0, ln)], dst, in_sem_b)
            for (p_, off, ln), buf in batch:
                dst = out_hbm.at[pl.ds(obase + p_ * N_pad + off, ln)]
                pltpu.make_async_copy(buf.at[pl.ds(0, ln)], dst,
                                      in_sem_b).wait()

    return pl.kernel(
        body,
        out_type=jax.ShapeDtypeStruct((_NC * 2 * N_pad,), jnp.float32),
        mesh=mesh,
        scratch_types=(
            [pltpu.VMEM((N,), jnp.float32)]
            + 2 * [pltpu.VMEM((_NBLK, 2, _BLK), jnp.int32),
                   pltpu.VMEM((_CH,), jnp.float32),
                   pltpu.VMEM((_CH,), jnp.int32),
                   pltpu.VMEM((_CH,), jnp.float32),
                   pltpu.VMEM((_CH,), jnp.float32)]
            + [pltpu.VMEM_SHARED((N_pad,), jnp.float32),
               pltpu.VMEM_SHARED((N_pad,), jnp.float32),
               pltpu.SemaphoreType.DMA,
               pltpu.SemaphoreType.DMA,
               pltpu.SemaphoreType.DMA,
               pltpu.SemaphoreType.DMA]
        ),
        compiler_params=pltpu.CompilerParams(needs_layout_passes=False),
        name="edge_scatter_sc",
    )


def _node_tc_body(N, R,
                  x_ref, agg_ref, w1, b1, w2, b2, w3, b3, w4, b4, out_ref):
    xv = x_ref[...]
    s1 = agg_ref[0] + agg_ref[2]
    s2 = agg_ref[1] + agg_ref[3]
    gidx = (lax.broadcasted_iota(jnp.int32, (R, 128), 0) * 128
            + lax.broadcasted_iota(jnp.int32, (R, 128), 1))
    mask = gidx < N

    n1 = ((jnp.exp((s2 / 0.3038425 + s1) * _LOG_A)
           + jnp.exp(s1 * _LOG_B) / -0.7256157)
          * jnp.exp(xv * _LOG_C) + 0.12262904)
    t = s2 + (s1 + -3.283101 - xv / 0.79082423) * 0.31992579
    n1_n2 = 0.7872602 - jnp.sqrt(jnp.log(jnp.exp(t * _LOG_D) + 1.4462701))
    h0 = jnp.where(mask, n1, 0.0)
    h1 = jnp.where(mask, n1_n2 - n1, 0.0)
    ps1 = jnp.sum(h0)
    ps2 = jnp.sum(h1)

    a = jnp.maximum(ps1 * w1[0, 0] + ps2 * w1[1, 0] + b1[0], 0.0)
    b_ = jnp.maximum(ps1 * w1[0, 1] + ps2 * w1[1, 1] + b1[1], 0.0)
    a2 = jnp.maximum(a * w2[0, 0] + b_ * w2[1, 0] + b2[0], 0.0)
    b2_ = jnp.maximum(a * w2[0, 1] + b_ * w2[1, 1] + b2[1], 0.0)
    a3 = jnp.maximum(a2 * w3[0, 0] + b2_ * w3[1, 0] + b3[0], 0.0)
    b3_ = jnp.maximum(a2 * w3[0, 1] + b2_ * w3[1, 1] + b3[1], 0.0)
    o1 = a3 * w4[0, 1] + b3_ * w4[1, 1] + b4[1]
    o0 = ((ps2 / -0.18032177 + ps1 * 2.2054937
           + jnp.abs(ps2 * 0.9565731 + ps1 * 0.8225316))
          * 0.00046277698 + -0.24634261)

    r8 = lax.broadcasted_iota(jnp.int32, (8, 128), 0)
    c8 = lax.broadcasted_iota(jnp.int32, (8, 128), 1)
    out_ref[...] = jnp.where(
        (r8 == 0) & (c8 == 0), o0,
        jnp.where((r8 == 0) & (c8 == 1), o1, 0.0))


@functools.lru_cache(maxsize=None)
def _node_tc_kernel(N, R):
    smem = pl.BlockSpec(memory_space=pltpu.SMEM)
    return pl.pallas_call(
        functools.partial(_node_tc_body, N, R),
        out_shape=jax.ShapeDtypeStruct((8, 128), jnp.float32),
        in_specs=[pl.BlockSpec(memory_space=pltpu.VMEM),
                  pl.BlockSpec(memory_space=pltpu.VMEM),
                  smem, smem, smem, smem, smem, smem, smem, smem],
        out_specs=pl.BlockSpec(memory_space=pltpu.VMEM),
        name="node_pool_tc",
    )


def kernel(x, edge_attr, e_W1, e_b1, e_W2, e_b2, n_W1, n_b1, n_W2, n_b2,
           o_W1, o_b1, o_W2, o_b2, o_W3, o_b3, o_W4, o_b4, edge_index, batch):
    N = x.shape[0]
    E = edge_index.shape[1]
    N_pad = ((N + 127) // 128) * 128
    x_flat = x[:, 0]

    ei3 = edge_index.reshape(2, E // _BLK, _BLK).transpose(1, 0, 2)
    db = edge_attr[:, 0] + edge_attr[:, 2]

    aggs = _edge_sc_kernel(N, E, N_pad)(x_flat, ei3, db)

    R = N_pad // 128
    x_pad = jnp.pad(x_flat, (0, N_pad - N)).reshape(R, 128)
    agg4 = aggs.reshape(4, R, 128)
    out8 = _node_tc_kernel(N, R)(
        x_pad, agg4, o_W1, o_b1, o_W2, o_b2, o_W3, o_b3, o_W4, o_b4)
    return out8[0:1, 0:2]


# speedup vs baseline: 424.4561x; 1.1034x over previous
import functools

import jax
import jax.numpy as jnp
import numpy as np
from jax import lax
from jax.experimental import pallas as pl
from jax.experimental.pallas import tpu as pltpu
from jax.experimental.pallas import tpu_sc as plsc

_NC = 2
_NS = 16
_NW = _NC * _NS
_BLK = 128
_NBLK = 12
_CH = _NBLK * _BLK

_LOG_A = np.float32(np.log(0.7660379))
_LOG_B = np.float32(np.log(0.12117091))
_LOG_C = np.float32(np.log(1.2125463))
_LOG_D = np.float32(np.log(0.1562228))


@functools.lru_cache(maxsize=None)
def _edge_sc_kernel(N, E, N_pad):
    NB = E // _BLK
    q, r = divmod(NB, _NW)
    NCH = -(-(q + (1 if r else 0)) // _NBLK)
    assert NCH % 2 == 0 and NCH >= 4, NCH
    NP16 = N_pad // _NS

    mesh = plsc.VectorSubcoreMesh(core_axis_name="c", subcore_axis_name="s")

    pieces = [(i * _CH, _CH) for i in range(NP16 // _CH)]
    if NP16 % _CH:
        pieces.append((NP16 - NP16 % _CH, NP16 % _CH))

    def body(x_hbm, ei_hbm, db_hbm, out_hbm,
             x_v, ei_a, db_a, col_a, e0_a, e1_a,
             ei_b, db_b, col_b, e0_b, e1_b,
             agg0_s, agg1_s, in_sem_a, in_sem_b, sc_sem_a, sc_sem_b):
        c = lax.axis_index("c")
        s = lax.axis_index("s")
        wid = c * _NS + s
        start_blk = q * wid + jnp.minimum(wid, r)
        nblk_w = q + jnp.where(wid < r, 1, 0)
        last_base = start_blk + nblk_w - _NBLK
        seta = (ei_a, db_a, col_a, e0_a, e1_a, in_sem_a, sc_sem_a)
        setb = (ei_b, db_b, col_b, e0_b, e1_b, in_sem_b, sc_sem_b)

        def chunk_base(ci):
            return jnp.minimum(start_blk + ci * _NBLK, last_base)

        def fire_inputs(ci, st):
            ei_v, db_v, _, _, _, isem, _ = st
            base = chunk_base(ci)
            pltpu.async_copy(ei_hbm.at[pl.ds(base, _NBLK), :, :], ei_v, isem)
            pltpu.async_copy(db_hbm.at[pl.ds(base * _BLK, _CH)], db_v, isem)

        def wait_inputs(ci, st):
            ei_v, db_v, _, _, _, isem, _ = st
            base = chunk_base(ci)
            pltpu.make_async_copy(
                ei_hbm.at[pl.ds(base, _NBLK), :, :], ei_v, isem).wait()
            pltpu.make_async_copy(
                db_hbm.at[pl.ds(base * _BLK, _CH)], db_v, isem).wait()

        def compute(ci, st):
            ei_v, db_v, col_v, e0_v, e1_v, _, _ = st
            base = chunk_base(ci)
            vstart = start_blk + ci * _NBLK

            @plsc.parallel_loop(0, _CH, step=16)
            def vec_body(p):
                blk = p // _BLK
                l = p - blk * _BLK
                rows = ei_v[blk, 0, pl.ds(l, 16)]
                cols = ei_v[blk, 1, pl.ds(l, 16)]
                vi = plsc.load_gather(x_v, [rows])
                vj = plsc.load_gather(x_v, [cols])
                db = db_v[pl.ds(p, 16)]
                valid = base + blk >= vstart
                e0 = (jnp.abs((vi / 0.9484139 - (vj - 0.2123214)) * -1.3248432)
                      + (db - 1.7348461 + vj) * -0.12084719)
                e1 = (jnp.abs((vi - vj * 1.0584362) * 1.5344211 + 0.45368108)
                      + (vi - vj * 1.0239582) * 1.931712 + 0.546892)
                col_v[pl.ds(p, 16)] = cols
                e0_v[pl.ds(p, 16)] = jnp.where(valid, e0, 0.0)
                e1_v[pl.ds(p, 16)] = jnp.where(valid, e1, 0.0)

        def fire_scatter(st):
            _, _, col_v, e0_v, e1_v, _, ssem = st
            pltpu.async_copy(e0_v, agg0_s.at[col_v], ssem, add=True)
            pltpu.async_copy(e1_v, agg1_s.at[col_v], ssem, add=True)

        def drain_scatter(st):
            _, _, col_v, e0_v, e1_v, _, ssem = st
            pltpu.make_async_copy(e0_v, agg0_s.at[col_v], ssem).wait()
            pltpu.make_async_copy(e1_v, agg1_s.at[col_v], ssem).wait()

        def zbody(i, carry):
            e0_a[pl.ds(i * 16, 16)] = jnp.zeros((16,), jnp.float32)
            return carry
        lax.fori_loop(0, _CH // 16, zbody, 0)
        pltpu.async_copy(x_hbm, x_v, in_sem_a)
        fire_inputs(0, seta)
        fire_inputs(1, setb)
        for off, ln in pieces:
            pltpu.async_copy(e0_a.at[pl.ds(0, ln)],
                             agg0_s.at[pl.ds(s * NP16 + off, ln)], sc_sem_a)
            pltpu.async_copy(e0_a.at[pl.ds(0, ln)],
                             agg1_s.at[pl.ds(s * NP16 + off, ln)], sc_sem_a)
        for off, ln in pieces:
            pltpu.make_async_copy(
                e0_a.at[pl.ds(0, ln)],
                agg0_s.at[pl.ds(s * NP16 + off, ln)], sc_sem_a).wait()
            pltpu.make_async_copy(
                e0_a.at[pl.ds(0, ln)],
                agg1_s.at[pl.ds(s * NP16 + off, ln)], sc_sem_a).wait()
        plsc.subcore_barrier()

        pltpu.make_async_copy(x_hbm, x_v, in_sem_a).wait()
        wait_inputs(0, seta)
        compute(0, seta)
        fire_scatter(seta)

        def pair_body(g, carry):
            ci1 = 1 + 2 * g
            wait_inputs(ci1, setb)
            compute(ci1, setb)
            drain_scatter(seta)
            fire_inputs(ci1 + 1, seta)
            fire_scatter(setb)
            ci2 = ci1 + 1
            wait_inputs(ci2, seta)
            compute(ci2, seta)
            drain_scatter(setb)
            fire_inputs(ci2 + 1, setb)
            fire_scatter(seta)
            return carry
        lax.fori_loop(0, (NCH - 2) // 2, pair_body, 0)

        ci = NCH - 1
        wait_inputs(ci, setb)
        compute(ci, setb)
        drain_scatter(seta)
        fire_scatter(setb)
        drain_scatter(setb)

        plsc.subcore_barrier()
        obase = c * 2 * N_pad + s * NP16
        stage = [e0_a, e1_a, db_a, e0_b, e1_b, db_b]
        tasks = [(p_, off, ln) for p_ in (0, 1) for off, ln in pieces]
        for i0 in range(0, len(tasks), len(stage)):
            batch = list(zip(tasks[i0:i0 + len(stage)], stage))
            for (p_, off, ln), buf in batch:
                src = (agg0_s if p_ == 0 else agg1_s)
                pltpu.async_copy(src.at[pl.ds(s * NP16 + off, ln)],
                                 buf.at[pl.ds(0, ln)], in_sem_a)
            for (p_, off, ln), buf in batch:
                src = (agg0_s if p_ == 0 else agg1_s)
                pltpu.make_async_copy(src.at[pl.ds(s * NP16 + off, ln)],
                                      buf.at[pl.ds(0, ln)], in_sem_a).wait()
            for (p_, off, ln), buf in batch:
                dst = out_hbm.at[pl.ds(obase + p_ * N_pad + off, ln)]
                pltpu.async_copy(buf.at[pl.ds(0, ln)], dst, in_sem_b)
            for (p_, off, ln), buf in batch:
                dst = out_hbm.at[pl.ds(obase + p_ * N_pad + off, ln)]
                pltpu.make_async_copy(buf.at[pl.ds(0, ln)], dst,
                                      in_sem_b).wait()

    return pl.kernel(
        body,
        out_type=jax.ShapeDtypeStruct((_NC * 2 * N_pad,), jnp.float32),
        mesh=mesh,
        scratch_types=(
            [pltpu.VMEM((N,), jnp.float32)]
            + 2 * [pltpu.VMEM((_NBLK, 2, _BLK), jnp.int32),
                   pltpu.VMEM((_CH,), jnp.float32),
                   pltpu.VMEM((_CH,), jnp.int32),
                   pltpu.VMEM((_CH,), jnp.float32),
                   pltpu.VMEM((_CH,), jnp.float32)]
            + [pltpu.VMEM_SHARED((N_pad,), jnp.float32),
               pltpu.VMEM_SHARED((N_pad,), jnp.float32),
               pltpu.SemaphoreType.DMA,
               pltpu.SemaphoreType.DMA,
               pltpu.SemaphoreType.DMA,
               pltpu.SemaphoreType.DMA]
        ),
        compiler_params=pltpu.CompilerParams(needs_layout_passes=False),
        name="edge_scatter_sc",
    )


def _node_tc_body(N, R,
                  x_ref, agg_ref, w1, b1, w2, b2, w3, b3, w4, b4, out_ref):
    xv = x_ref[...]
    s1 = agg_ref[0] + agg_ref[2]
    s2 = agg_ref[1] + agg_ref[3]
    gidx = (lax.broadcasted_iota(jnp.int32, (R, 128), 0) * 128
            + lax.broadcasted_iota(jnp.int32, (R, 128), 1))
    mask = gidx < N

    n1 = ((jnp.exp((s2 / 0.3038425 + s1) * _LOG_A)
           + jnp.exp(s1 * _LOG_B) / -0.7256157)
          * jnp.exp(xv * _LOG_C) + 0.12262904)
    t = s2 + (s1 + -3.283101 - xv / 0.79082423) * 0.31992579
    n1_n2 = 0.7872602 - jnp.sqrt(jnp.log(jnp.exp(t * _LOG_D) + 1.4462701))
    h0 = jnp.where(mask, n1, 0.0)
    h1 = jnp.where(mask, n1_n2 - n1, 0.0)
    ps1 = jnp.sum(h0)
    ps2 = jnp.sum(h1)

    a = jnp.maximum(ps1 * w1[0, 0] + ps2 * w1[1, 0] + b1[0], 0.0)
    b_ = jnp.maximum(ps1 * w1[0, 1] + ps2 * w1[1, 1] + b1[1], 0.0)
    a2 = jnp.maximum(a * w2[0, 0] + b_ * w2[1, 0] + b2[0], 0.0)
    b2_ = jnp.maximum(a * w2[0, 1] + b_ * w2[1, 1] + b2[1], 0.0)
    a3 = jnp.maximum(a2 * w3[0, 0] + b2_ * w3[1, 0] + b3[0], 0.0)
    b3_ = jnp.maximum(a2 * w3[0, 1] + b2_ * w3[1, 1] + b3[1], 0.0)
    o1 = a3 * w4[0, 1] + b3_ * w4[1, 1] + b4[1]
    o0 = ((ps2 / -0.18032177 + ps1 * 2.2054937
           + jnp.abs(ps2 * 0.9565731 + ps1 * 0.8225316))
          * 0.00046277698 + -0.24634261)

    r8 = lax.broadcasted_iota(jnp.int32, (8, 128), 0)
    c8 = lax.broadcasted_iota(jnp.int32, (8, 128), 1)
    out_ref[...] = jnp.where(
        (r8 == 0) & (c8 == 0), o0,
        jnp.where((r8 == 0) & (c8 == 1), o1, 0.0))


@functools.lru_cache(maxsize=None)
def _node_tc_kernel(N, R):
    smem = pl.BlockSpec(memory_space=pltpu.SMEM)
    return pl.pallas_call(
        functools.partial(_node_tc_body, N, R),
        out_shape=jax.ShapeDtypeStruct((8, 128), jnp.float32),
        in_specs=[pl.BlockSpec(memory_space=pltpu.VMEM),
                  pl.BlockSpec(memory_space=pltpu.VMEM),
                  smem, smem, smem, smem, smem, smem, smem, smem],
        out_specs=pl.BlockSpec(memory_space=pltpu.VMEM),
        name="node_pool_tc",
    )


def kernel(x, edge_attr, e_W1, e_b1, e_W2, e_b2, n_W1, n_b1, n_W2, n_b2,
           o_W1, o_b1, o_W2, o_b2, o_W3, o_b3, o_W4, o_b4, edge_index, batch):
    N = x.shape[0]
    E = edge_index.shape[1]
    N_pad = ((N + 127) // 128) * 128
    x_flat = x[:, 0]

    ei3 = edge_index.reshape(2, E // _BLK, _BLK).transpose(1, 0, 2)
    db = edge_attr[:, 0] + edge_attr[:, 2]

    aggs = _edge_sc_kernel(N, E, N_pad)(x_flat, ei3, db)

    R = N_pad // 128
    x_pad = jnp.pad(x_flat, (0, N_pad - N)).reshape(R, 128)
    agg4 = aggs.reshape(4, R, 128)
    out8 = _node_tc_kernel(N, R)(
        x_pad, agg4, o_W1, o_b1, o_W2, o_b2, o_W3, o_b3, o_W4, o_b4)
    return out8[0:1, 0:2]
